# baseline (speedup 1.0000x reference)
import functools

import jax
import jax.numpy as jnp
from jax import lax
from jax.experimental import pallas as pl
from jax.experimental.pallas import tpu as pltpu
from jax.experimental.pallas import tpu_sc as plsc

_N = 10000
_E = 320000
_D = 128
_CE = 1280
_NCHUNK = _E // _CE
_CPT = 4
_L = 16
_NEG = float("-inf")

_GATHER_DN = lax.GatherDimensionNumbers(
    offset_dims=(), collapsed_slice_dims=(0,), start_index_map=(0,)
)


def _shift(v, idx):
    return lax.gather(
        v, idx[:, None], _GATHER_DN, slice_sizes=(1,),
        mode=lax.GatherScatterMode.PROMISE_IN_BOUNDS,
    )


def _seg_max_sc(q8, ei):
    mesh = plsc.VectorSubcoreMesh(core_axis_name="c", subcore_axis_name="s")

    @functools.partial(
        pl.kernel,
        mesh=mesh,
        compiler_params=pltpu.CompilerParams(needs_layout_passes=False),
        out_type=jax.ShapeDtypeStruct((32, 8, _N), jnp.float32),
        scratch_types=[
            [pltpu.VMEM((_N,), jnp.float32)] * _CPT,
            [pltpu.VMEM((_N,), jnp.float32)] * _CPT,
            pltpu.VMEM((2, 2, _CE), jnp.int32),
            pltpu.SemaphoreType.DMA,
            pltpu.SemaphoreType.DMA,
            pltpu.SemaphoreType.DMA,
        ],
    )
    def k(q_hbm, ei_hbm, s_hbm, qt_vs, acc_vs, idx_v, semq, sem0, sem1):
        wid = lax.axis_index("s") * 2 + lax.axis_index("c")
        cqs = [
            pltpu.async_copy(q_hbm.at[wid, c, :], qt_vs[c], semq)
            for c in range(_CPT)
        ]
        pltpu.async_copy(ei_hbm.at[:, pl.ds(0, _CE)], idx_v.at[0], sem0)
        pltpu.async_copy(ei_hbm.at[:, pl.ds(_CE, _CE)], idx_v.at[1], sem1)

        iota = lax.iota(jnp.int32, _L)
        neg = jnp.full((_L,), _NEG, jnp.float32)

        def init_body(i, _):
            for c in range(_CPT):
                acc_vs[c][pl.ds(i * _L, _L)] = neg
            return 0

        lax.fori_loop(0, _N // _L, init_body, 0, unroll=4)
        for cq in cqs:
            cq.wait()

        shift_idx = [jnp.maximum(iota - s, 0) for s in (1, 2, 4, 8)]
        shift_ok = [iota >= s for s in (1, 2, 4, 8)]
        next_idx = jnp.minimum(iota + 1, _L - 1)
        is_last = iota == _L - 1

        def process(p):
            def group(g, _):
                b = g * _L
                s = idx_v[p, 0, pl.ds(b, _L)]
                d = idx_v[p, 1, pl.ds(b, _L)]
                d_s, s_s = plsc.sort_key_val(d, s)
                eqs = [
                    (_shift(d_s, si) == d_s) & ok
                    for si, ok in zip(shift_idx, shift_ok)
                ]
                leader = (_shift(d_s, next_idx) != d_s) | is_last
                for c in range(_CPT):
                    v = plsc.load_gather(qt_vs[c], [s_s])
                    for si, eq in zip(shift_idx, eqs):
                        v = jnp.maximum(v, jnp.where(eq, _shift(v, si), _NEG))
                    cur = plsc.load_gather(acc_vs[c], [d_s])
                    plsc.store_scatter(
                        acc_vs[c], [d_s], jnp.maximum(cur, v), mask=leader
                    )
                return 0

            lax.fori_loop(0, _CE // _L, group, 0, unroll=4)

        def pair(j, _):
            pltpu.make_async_copy(
                ei_hbm.at[:, pl.ds(0, _CE)], idx_v.at[0], sem0
            ).wait()
            process(0)
            off0 = jnp.minimum((2 * j + 2) * _CE, _E - _CE)
            pltpu.async_copy(ei_hbm.at[:, pl.ds(off0, _CE)], idx_v.at[0], sem0)
            pltpu.make_async_copy(
                ei_hbm.at[:, pl.ds(0, _CE)], idx_v.at[1], sem1
            ).wait()
            process(1)
            off1 = jnp.minimum((2 * j + 3) * _CE, _E - _CE)
            pltpu.async_copy(ei_hbm.at[:, pl.ds(off1, _CE)], idx_v.at[1], sem1)
            return 0

        lax.fori_loop(0, _NCHUNK // 2, pair, 0)
        pltpu.make_async_copy(ei_hbm.at[:, pl.ds(0, _CE)], idx_v.at[0], sem0).wait()
        pltpu.make_async_copy(ei_hbm.at[:, pl.ds(0, _CE)], idx_v.at[1], sem1).wait()
        for c in range(_CPT):
            pltpu.sync_copy(acc_vs[c], s_hbm.at[wid, c, :])

    return k(q8, ei)


def _mm1(x_t, w_t, b2d):
    d, n = x_t.shape

    def body(w_ref, x_ref, b_ref, p_ref, q_ref):
        o = (
            jnp.dot(w_ref[...], x_ref[...], preferred_element_type=jnp.float32)
            + b_ref[...]
        )
        p_ref[...] = o[:d]
        q_ref[:, 0:_CPT, :] = o[d:].reshape(32, _CPT, n)

    return pl.pallas_call(
        body,
        out_shape=[
            jax.ShapeDtypeStruct((d, n), jnp.float32),
            jax.ShapeDtypeStruct((32, 8, n), jnp.float32),
        ],
    )(w_t, x_t, b2d)


def _mm2(p1t, s8, a2d, w_t, b2d):
    d, n = p1t.shape

    def body(p_ref, s8_ref, a_ref, w_ref, b_ref, p2_ref, q2_ref):
        s = s8_ref[:, 0:_CPT, :].reshape(d, n)
        z = jnp.where(jnp.isfinite(s), p_ref[...] + s, 0.0)
        h = jnp.where(z >= 0, z, a_ref[...] * z)
        o = (
            jnp.dot(w_ref[...], h, preferred_element_type=jnp.float32)
            + b_ref[...]
        )
        p2_ref[...] = o[:d]
        q2_ref[:, 0:_CPT, :] = o[d:].reshape(32, _CPT, n)

    return pl.pallas_call(
        body,
        out_shape=[
            jax.ShapeDtypeStruct((d, n), jnp.float32),
            jax.ShapeDtypeStruct((32, 8, n), jnp.float32),
        ],
    )(p1t, s8, a2d, w_t, b2d)


def _combine(p2t, s8):
    d, n = p2t.shape

    def body(p_ref, s8_ref, o_ref):
        s = s8_ref[:, 0:_CPT, :].reshape(d, n)
        o_ref[...] = jnp.where(jnp.isfinite(s), p_ref[...] + s, 0.0)

    return pl.pallas_call(
        body,
        out_shape=jax.ShapeDtypeStruct((d, n), jnp.float32),
    )(p2t, s8)


@jax.jit
def kernel(x1, edge_index1, x2, edge_index2, W1, b1, prelu_a, W2, b2):
    d = x1.shape[1]
    dh = W1.shape[1]
    wc1_t = jnp.concatenate([W1[:d] - W1[d:], W1[d:]], axis=1).T
    bc1 = jnp.concatenate([b1, jnp.zeros_like(b1)]).reshape(2 * d, 1)
    wc2_t = jnp.concatenate([W2[:dh] - W2[dh:], W2[dh:]], axis=1).T
    bc2 = jnp.concatenate([b2, jnp.zeros_like(b2)]).reshape(2 * dh, 1)
    a2d = prelu_a.reshape(dh, 1)

    def tower(x, ei):
        p1t, q8 = _mm1(x.T, wc1_t, bc1)
        s8_1 = _seg_max_sc(q8, ei)
        p2t, q8_2 = _mm2(p1t, s8_1, a2d, wc2_t, bc2)
        s8_2 = _seg_max_sc(q8_2, ei)
        return _combine(p2t, s8_2).T

    return tower(x1, edge_index1), tower(x2, edge_index2)


# speedup vs baseline: 2.6022x; 2.6022x over previous
import functools

import jax
import jax.numpy as jnp
from jax import lax
from jax.experimental import pallas as pl
from jax.experimental.pallas import tpu as pltpu
from jax.experimental.pallas import tpu_sc as plsc

_N = 10000
_E = 320000
_D = 128
_CE = 1280
_NCHUNK = _E // _CE
_CPT = 4
_L = 16
_K = 4
_NEG = float("-inf")

_GATHER_DN = lax.GatherDimensionNumbers(
    offset_dims=(), collapsed_slice_dims=(0,), start_index_map=(0,)
)


def _shift(v, idx):
    return lax.gather(
        v, idx[:, None], _GATHER_DN, slice_sizes=(1,),
        mode=lax.GatherScatterMode.PROMISE_IN_BOUNDS,
    )


def _seg_max_sc(q8, ei):
    mesh = plsc.VectorSubcoreMesh(core_axis_name="c", subcore_axis_name="s")

    @functools.partial(
        pl.kernel,
        mesh=mesh,
        compiler_params=pltpu.CompilerParams(needs_layout_passes=False),
        out_type=jax.ShapeDtypeStruct((32, 8, _N), jnp.float32),
        scratch_types=[
            [pltpu.VMEM((_N,), jnp.float32)] * _CPT,
            [pltpu.VMEM((_N,), jnp.float32)] * _CPT,
            pltpu.VMEM((2, 2, _CE), jnp.int32),
            pltpu.SemaphoreType.DMA,
            pltpu.SemaphoreType.DMA,
            pltpu.SemaphoreType.DMA,
        ],
    )
    def k(q_hbm, ei_hbm, s_hbm, qt_vs, acc_vs, idx_v, semq, sem0, sem1):
        wid = lax.axis_index("s") * 2 + lax.axis_index("c")
        cqs = [
            pltpu.async_copy(q_hbm.at[wid, c, :], qt_vs[c], semq)
            for c in range(_CPT)
        ]
        pltpu.async_copy(ei_hbm.at[:, pl.ds(0, _CE)], idx_v.at[0], sem0)
        pltpu.async_copy(ei_hbm.at[:, pl.ds(_CE, _CE)], idx_v.at[1], sem1)

        iota = lax.iota(jnp.int32, _L)
        neg = jnp.full((_L,), _NEG, jnp.float32)

        def init_body(i, _):
            for c in range(_CPT):
                acc_vs[c][pl.ds(i * _L, _L)] = neg
            return 0

        lax.fori_loop(0, _N // _L, init_body, 0, unroll=4)
        for cq in cqs:
            cq.wait()

        shift_idx = [jnp.maximum(iota - s, 0) for s in (1, 2, 4, 8)]
        shift_ok = [iota >= s for s in (1, 2, 4, 8)]
        next_idx = jnp.minimum(iota + 1, _L - 1)
        is_last = iota == _L - 1

        def process(p):
            def block(t, _):
                per_group = []
                for k in range(_K):
                    b = (t * _K + k) * _L
                    s = idx_v[p, 0, pl.ds(b, _L)]
                    d = idx_v[p, 1, pl.ds(b, _L)]
                    d_s, s_s = plsc.sort_key_val(d, s)
                    eqs = [
                        (_shift(d_s, si) == d_s) & ok
                        for si, ok in zip(shift_idx, shift_ok)
                    ]
                    leader = (_shift(d_s, next_idx) != d_s) | is_last
                    vals = []
                    for c in range(_CPT):
                        v = plsc.load_gather(qt_vs[c], [s_s])
                        for si, eq in zip(shift_idx, eqs):
                            v = jnp.maximum(
                                v, jnp.where(eq, _shift(v, si), _NEG)
                            )
                        vals.append(v)
                    per_group.append((d_s, leader, vals))
                for c in range(_CPT):
                    for d_s, leader, vals in per_group:
                        cur = plsc.load_gather(acc_vs[c], [d_s])
                        plsc.store_scatter(
                            acc_vs[c], [d_s], jnp.maximum(cur, vals[c]),
                            mask=leader,
                        )
                return 0

            lax.fori_loop(0, _CE // (_L * _K), block, 0)

        def pair(j, _):
            pltpu.make_async_copy(
                ei_hbm.at[:, pl.ds(0, _CE)], idx_v.at[0], sem0
            ).wait()
            process(0)
            off0 = jnp.minimum((2 * j + 2) * _CE, _E - _CE)
            pltpu.async_copy(ei_hbm.at[:, pl.ds(off0, _CE)], idx_v.at[0], sem0)
            pltpu.make_async_copy(
                ei_hbm.at[:, pl.ds(0, _CE)], idx_v.at[1], sem1
            ).wait()
            process(1)
            off1 = jnp.minimum((2 * j + 3) * _CE, _E - _CE)
            pltpu.async_copy(ei_hbm.at[:, pl.ds(off1, _CE)], idx_v.at[1], sem1)
            return 0

        lax.fori_loop(0, _NCHUNK // 2, pair, 0)
        pltpu.make_async_copy(ei_hbm.at[:, pl.ds(0, _CE)], idx_v.at[0], sem0).wait()
        pltpu.make_async_copy(ei_hbm.at[:, pl.ds(0, _CE)], idx_v.at[1], sem1).wait()
        for c in range(_CPT):
            pltpu.sync_copy(acc_vs[c], s_hbm.at[wid, c, :])

    return k(q8, ei)


def _mm1(x_t, w_t, b2d):
    d, n = x_t.shape

    def body(w_ref, x_ref, b_ref, p_ref, q_ref):
        o = (
            jnp.dot(w_ref[...], x_ref[...], preferred_element_type=jnp.float32)
            + b_ref[...]
        )
        p_ref[...] = o[:d]
        q_ref[:, 0:_CPT, :] = o[d:].reshape(32, _CPT, n)

    return pl.pallas_call(
        body,
        out_shape=[
            jax.ShapeDtypeStruct((d, n), jnp.float32),
            jax.ShapeDtypeStruct((32, 8, n), jnp.float32),
        ],
    )(w_t, x_t, b2d)


def _mm2(p1t, s8, a2d, w_t, b2d):
    d, n = p1t.shape

    def body(p_ref, s8_ref, a_ref, w_ref, b_ref, p2_ref, q2_ref):
        s = s8_ref[:, 0:_CPT, :].reshape(d, n)
        z = jnp.where(jnp.isfinite(s), p_ref[...] + s, 0.0)
        h = jnp.where(z >= 0, z, a_ref[...] * z)
        o = (
            jnp.dot(w_ref[...], h, preferred_element_type=jnp.float32)
            + b_ref[...]
        )
        p2_ref[...] = o[:d]
        q2_ref[:, 0:_CPT, :] = o[d:].reshape(32, _CPT, n)

    return pl.pallas_call(
        body,
        out_shape=[
            jax.ShapeDtypeStruct((d, n), jnp.float32),
            jax.ShapeDtypeStruct((32, 8, n), jnp.float32),
        ],
    )(p1t, s8, a2d, w_t, b2d)


def _combine(p2t, s8):
    d, n = p2t.shape

    def body(p_ref, s8_ref, o_ref):
        s = s8_ref[:, 0:_CPT, :].reshape(d, n)
        o_ref[...] = jnp.where(jnp.isfinite(s), p_ref[...] + s, 0.0)

    return pl.pallas_call(
        body,
        out_shape=jax.ShapeDtypeStruct((d, n), jnp.float32),
    )(p2t, s8)


@jax.jit
def kernel(x1, edge_index1, x2, edge_index2, W1, b1, prelu_a, W2, b2):
    d = x1.shape[1]
    dh = W1.shape[1]
    wc1_t = jnp.concatenate([W1[:d] - W1[d:], W1[d:]], axis=1).T
    bc1 = jnp.concatenate([b1, jnp.zeros_like(b1)]).reshape(2 * d, 1)
    wc2_t = jnp.concatenate([W2[:dh] - W2[dh:], W2[dh:]], axis=1).T
    bc2 = jnp.concatenate([b2, jnp.zeros_like(b2)]).reshape(2 * dh, 1)
    a2d = prelu_a.reshape(dh, 1)

    def tower(x, ei):
        p1t, q8 = _mm1(x.T, wc1_t, bc1)
        s8_1 = _seg_max_sc(q8, ei)
        p2t, q8_2 = _mm2(p1t, s8_1, a2d, wc2_t, bc2)
        s8_2 = _seg_max_sc(q8_2, ei)
        return _combine(p2t, s8_2).T

    return tower(x1, edge_index1), tower(x2, edge_index2)


# speedup vs baseline: 2.6342x; 1.0123x over previous
import functools

import jax
import jax.numpy as jnp
from jax import lax
from jax.experimental import pallas as pl
from jax.experimental.pallas import tpu as pltpu
from jax.experimental.pallas import tpu_sc as plsc

_N = 10000
_E = 320000
_D = 128
_CE = 1280
_NCHUNK = _E // _CE
_CPT = 4
_L = 16
_K = 8
_NEG = float("-inf")

_GATHER_DN = lax.GatherDimensionNumbers(
    offset_dims=(), collapsed_slice_dims=(0,), start_index_map=(0,)
)


def _shift(v, idx):
    return lax.gather(
        v, idx[:, None], _GATHER_DN, slice_sizes=(1,),
        mode=lax.GatherScatterMode.PROMISE_IN_BOUNDS,
    )


def _seg_max_sc(q8, ei):
    mesh = plsc.VectorSubcoreMesh(core_axis_name="c", subcore_axis_name="s")

    @functools.partial(
        pl.kernel,
        mesh=mesh,
        compiler_params=pltpu.CompilerParams(needs_layout_passes=False),
        out_type=jax.ShapeDtypeStruct((32, 8, _N), jnp.float32),
        scratch_types=[
            [pltpu.VMEM((_N,), jnp.float32)] * _CPT,
            [pltpu.VMEM((_N,), jnp.float32)] * _CPT,
            pltpu.VMEM((2, 2, _CE), jnp.int32),
            pltpu.SemaphoreType.DMA,
            pltpu.SemaphoreType.DMA,
            pltpu.SemaphoreType.DMA,
        ],
    )
    def k(q_hbm, ei_hbm, s_hbm, qt_vs, acc_vs, idx_v, semq, sem0, sem1):
        wid = lax.axis_index("s") * 2 + lax.axis_index("c")
        cqs = [
            pltpu.async_copy(q_hbm.at[wid, c, :], qt_vs[c], semq)
            for c in range(_CPT)
        ]
        pltpu.async_copy(ei_hbm.at[:, pl.ds(0, _CE)], idx_v.at[0], sem0)
        pltpu.async_copy(ei_hbm.at[:, pl.ds(_CE, _CE)], idx_v.at[1], sem1)

        iota = lax.iota(jnp.int32, _L)
        neg = jnp.full((_L,), _NEG, jnp.float32)

        def init_body(i, _):
            for c in range(_CPT):
                acc_vs[c][pl.ds(i * _L, _L)] = neg
            return 0

        lax.fori_loop(0, _N // _L, init_body, 0, unroll=4)
        for cq in cqs:
            cq.wait()

        shift_idx = [jnp.maximum(iota - s, 0) for s in (1, 2, 4, 8)]
        shift_ok = [iota >= s for s in (1, 2, 4, 8)]
        next_idx = jnp.minimum(iota + 1, _L - 1)
        is_last = iota == _L - 1

        def process(p):
            def block(t, _):
                per_group = []
                for k in range(_K):
                    b = (t * _K + k) * _L
                    s = idx_v[p, 0, pl.ds(b, _L)]
                    d = idx_v[p, 1, pl.ds(b, _L)]
                    d_s, s_s = plsc.sort_key_val(d, s)
                    eqs = [
                        (_shift(d_s, si) == d_s) & ok
                        for si, ok in zip(shift_idx, shift_ok)
                    ]
                    leader = (_shift(d_s, next_idx) != d_s) | is_last
                    vals = []
                    for c in range(_CPT):
                        v = plsc.load_gather(qt_vs[c], [s_s])
                        for si, eq in zip(shift_idx, eqs):
                            v = jnp.maximum(
                                v, jnp.where(eq, _shift(v, si), _NEG)
                            )
                        vals.append(v)
                    per_group.append((d_s, leader, vals))
                for c in range(_CPT):
                    for d_s, leader, vals in per_group:
                        cur = plsc.load_gather(acc_vs[c], [d_s])
                        plsc.store_scatter(
                            acc_vs[c], [d_s], jnp.maximum(cur, vals[c]),
                            mask=leader,
                        )
                return 0

            lax.fori_loop(0, _CE // (_L * _K), block, 0)

        def pair(j, _):
            pltpu.make_async_copy(
                ei_hbm.at[:, pl.ds(0, _CE)], idx_v.at[0], sem0
            ).wait()
            process(0)
            off0 = jnp.minimum((2 * j + 2) * _CE, _E - _CE)
            pltpu.async_copy(ei_hbm.at[:, pl.ds(off0, _CE)], idx_v.at[0], sem0)
            pltpu.make_async_copy(
                ei_hbm.at[:, pl.ds(0, _CE)], idx_v.at[1], sem1
            ).wait()
            process(1)
            off1 = jnp.minimum((2 * j + 3) * _CE, _E - _CE)
            pltpu.async_copy(ei_hbm.at[:, pl.ds(off1, _CE)], idx_v.at[1], sem1)
            return 0

        lax.fori_loop(0, _NCHUNK // 2, pair, 0)
        pltpu.make_async_copy(ei_hbm.at[:, pl.ds(0, _CE)], idx_v.at[0], sem0).wait()
        pltpu.make_async_copy(ei_hbm.at[:, pl.ds(0, _CE)], idx_v.at[1], sem1).wait()
        for c in range(_CPT):
            pltpu.sync_copy(acc_vs[c], s_hbm.at[wid, c, :])

    return k(q8, ei)


def _mm1(x_t, w_t, b2d):
    d, n = x_t.shape

    def body(w_ref, x_ref, b_ref, p_ref, q_ref):
        o = (
            jnp.dot(w_ref[...], x_ref[...], preferred_element_type=jnp.float32)
            + b_ref[...]
        )
        p_ref[...] = o[:d]
        q_ref[:, 0:_CPT, :] = o[d:].reshape(32, _CPT, n)

    return pl.pallas_call(
        body,
        out_shape=[
            jax.ShapeDtypeStruct((d, n), jnp.float32),
            jax.ShapeDtypeStruct((32, 8, n), jnp.float32),
        ],
    )(w_t, x_t, b2d)


def _mm2(p1t, s8, a2d, w_t, b2d):
    d, n = p1t.shape

    def body(p_ref, s8_ref, a_ref, w_ref, b_ref, p2_ref, q2_ref):
        s = s8_ref[:, 0:_CPT, :].reshape(d, n)
        z = jnp.where(jnp.isfinite(s), p_ref[...] + s, 0.0)
        h = jnp.where(z >= 0, z, a_ref[...] * z)
        o = (
            jnp.dot(w_ref[...], h, preferred_element_type=jnp.float32)
            + b_ref[...]
        )
        p2_ref[...] = o[:d]
        q2_ref[:, 0:_CPT, :] = o[d:].reshape(32, _CPT, n)

    return pl.pallas_call(
        body,
        out_shape=[
            jax.ShapeDtypeStruct((d, n), jnp.float32),
            jax.ShapeDtypeStruct((32, 8, n), jnp.float32),
        ],
    )(p1t, s8, a2d, w_t, b2d)


def _combine(p2t, s8):
    d, n = p2t.shape

    def body(p_ref, s8_ref, o_ref):
        s = s8_ref[:, 0:_CPT, :].reshape(d, n)
        o_ref[...] = jnp.where(jnp.isfinite(s), p_ref[...] + s, 0.0)

    return pl.pallas_call(
        body,
        out_shape=jax.ShapeDtypeStruct((d, n), jnp.float32),
    )(p2t, s8)


@jax.jit
def kernel(x1, edge_index1, x2, edge_index2, W1, b1, prelu_a, W2, b2):
    d = x1.shape[1]
    dh = W1.shape[1]
    wc1_t = jnp.concatenate([W1[:d] - W1[d:], W1[d:]], axis=1).T
    bc1 = jnp.concatenate([b1, jnp.zeros_like(b1)]).reshape(2 * d, 1)
    wc2_t = jnp.concatenate([W2[:dh] - W2[dh:], W2[dh:]], axis=1).T
    bc2 = jnp.concatenate([b2, jnp.zeros_like(b2)]).reshape(2 * dh, 1)
    a2d = prelu_a.reshape(dh, 1)

    def tower(x, ei):
        p1t, q8 = _mm1(x.T, wc1_t, bc1)
        s8_1 = _seg_max_sc(q8, ei)
        p2t, q8_2 = _mm2(p1t, s8_1, a2d, wc2_t, bc2)
        s8_2 = _seg_max_sc(q8_2, ei)
        return _combine(p2t, s8_2).T

    return tower(x1, edge_index1), tower(x2, edge_index2)
